# baseline (speedup 1.0000x reference)
import functools

import jax
import jax.numpy as jnp
from jax import lax
from jax.experimental import pallas as pl
from jax.experimental.pallas import tpu as pltpu
from jax.experimental.pallas import tpu_sc as plsc

_MESH = plsc.VectorSubcoreMesh(core_axis_name="c", subcore_axis_name="s")


@functools.partial(
    pl.kernel,
    out_type=jax.ShapeDtypeStruct((2, 10), jnp.float32),
    mesh=_MESH,
    compiler_params=pltpu.CompilerParams(needs_layout_passes=False),
    scratch_types=[
        pltpu.VMEM((16,), jnp.int32),
        pltpu.VMEM((16,), jnp.float32),
        pltpu.VMEM((2, 10), jnp.float32),
    ],
)
def _sc_update(update_hbm, index_hbm, params_hbm, out_hbm, idx_v, row_v, buf_v):
    cid = lax.axis_index("c")
    sid = lax.axis_index("s")

    @pl.when(jnp.logical_and(cid == 0, sid == 0))
    def _():
        idx_v[...] = jnp.zeros((16,), jnp.int32)
        pltpu.sync_copy(index_hbm, idx_v.at[pl.ds(0, 1)])
        pltpu.sync_copy(update_hbm, row_v.at[pl.ds(0, 10)])
        pltpu.sync_copy(params_hbm, buf_v)
        lane = lax.iota(jnp.int32, 16)
        mask = lane < 10
        r = jnp.cumsum(idx_v[...])
        plsc.store_scatter(buf_v, [r, lane], row_v[...], mask=mask)
        pltpu.sync_copy(buf_v, out_hbm)


def kernel(update, index, params):
    out = _sc_update(update.reshape(10), index, params.reshape(2, 10))
    return out.reshape(2, 1, 10)


# speedup vs baseline: 1.0506x; 1.0506x over previous
import functools

import jax
import jax.numpy as jnp
from jax import lax
from jax.experimental import pallas as pl
from jax.experimental.pallas import tpu as pltpu
from jax.experimental.pallas import tpu_sc as plsc

_MESH = plsc.ScalarSubcoreMesh(axis_name="c")


@functools.partial(
    pl.kernel,
    out_type=jax.ShapeDtypeStruct((2, 10), jnp.float32),
    mesh=_MESH,
    compiler_params=pltpu.CompilerParams(needs_layout_passes=False),
    scratch_types=[pltpu.SMEM((1,), jnp.int32)],
)
def _sc_update(update_hbm, index_hbm, params_hbm, out_hbm, idx_s):
    cid = lax.axis_index("c")

    @pl.when(cid == 0)
    def _():
        pltpu.sync_copy(index_hbm, idx_s)
        pltpu.sync_copy(params_hbm, out_hbm)
        i = idx_s[0]
        pltpu.sync_copy(update_hbm, out_hbm.at[pl.ds(i, 1)])


def kernel(update, index, params):
    out = _sc_update(update.reshape(1, 10), index, params.reshape(2, 10))
    return out.reshape(2, 1, 10)


# speedup vs baseline: 10.5189x; 10.0121x over previous
import jax
import jax.numpy as jnp
from jax import lax
from jax.experimental import pallas as pl
from jax.experimental.pallas import tpu as pltpu


def _tc_body(idx_ref, upd_ref, params_ref, out_ref):
    i = idx_ref[0]
    rows = lax.broadcasted_iota(jnp.int32, (2, 10), 0)
    out_ref[...] = jnp.where(rows == i, upd_ref[...], params_ref[...])


def kernel(update, index, params):
    out = pl.pallas_call(
        _tc_body,
        out_shape=jax.ShapeDtypeStruct((2, 10), jnp.float32),
        in_specs=[
            pl.BlockSpec(memory_space=pltpu.SMEM),
            pl.BlockSpec(memory_space=pltpu.VMEM),
            pl.BlockSpec(memory_space=pltpu.VMEM),
        ],
        out_specs=pl.BlockSpec(memory_space=pltpu.VMEM),
    )(index, update.reshape(1, 10), params.reshape(2, 10))
    return out.reshape(2, 1, 10)
